# baseline (speedup 1.0000x reference)
import functools

import jax
import jax.numpy as jnp
from jax import lax
from jax.experimental import pallas as pl
from jax.experimental.pallas import tpu as pltpu
from jax.experimental.pallas import tpu_sc as plsc

_NC = 2
_NS = 16
_NW = _NC * _NS
_CHUNK = 128
_BLK = 65536


def _tc_project_table(tableT, W, b):
    d, v = tableT.shape
    o = W.shape[0]
    grid = (v + _BLK - 1) // _BLK

    def body(t_ref, w_ref, b_ref, *o_refs):
        res = lax.dot_general(
            w_ref[...], t_ref[...], (((1,), (0,)), ((), ())),
            preferred_element_type=jnp.float32,
        )
        for j in range(o):
            o_refs[j][...] = res[j, :] + b_ref[j]

    return pl.pallas_call(
        body,
        grid=(grid,),
        in_specs=[
            pl.BlockSpec((d, _BLK), lambda i: (0, i)),
            pl.BlockSpec((o, d), lambda i: (0, 0)),
            pl.BlockSpec(memory_space=pltpu.SMEM),
        ],
        out_specs=tuple(pl.BlockSpec((_BLK,), lambda i: (i,)) for _ in range(o)),
        out_shape=tuple(
            jax.ShapeDtypeStruct((v,), jnp.float32) for _ in range(o)
        ),
    )(tableT, W, b)


def _sc_gather_cols(cols, idx2d):
    o = len(cols)
    n_rows, chunk = idx2d.shape
    batch = n_rows * chunk
    rows_per_w = n_rows // _NW
    b_per_w = batch // _NW

    mesh = plsc.VectorSubcoreMesh(core_axis_name="c", subcore_axis_name="s")

    @functools.partial(
        pl.kernel,
        mesh=mesh,
        compiler_params=pltpu.CompilerParams(use_tc_tiling_on_sc=False),
        out_type=jax.ShapeDtypeStruct((o, batch), jnp.float32),
        scratch_types=[
            pltpu.VMEM((rows_per_w, chunk), jnp.int32),
            pltpu.VMEM((o, b_per_w), jnp.float32),
            pltpu.SemaphoreType.DMA,
        ],
    )
    def gather(*refs):
        col_hbms = refs[:o]
        idx_hbm = refs[o]
        out_hbm = refs[o + 1]
        idx_v, vals_v, sem = refs[o + 2], refs[o + 3], refs[o + 4]
        wid = lax.axis_index("s") * _NC + lax.axis_index("c")
        row_base = wid * rows_per_w
        pltpu.sync_copy(idx_hbm.at[pl.ds(row_base, rows_per_w)], idx_v)
        copies = []
        for j in range(o):
            for i in range(rows_per_w):
                copies.append(
                    pltpu.async_copy(
                        col_hbms[j].at[idx_v.at[i]],
                        vals_v.at[j, pl.ds(i * chunk, chunk)],
                        sem,
                    )
                )
        for cp in copies:
            cp.wait()
        pltpu.sync_copy(
            vals_v, out_hbm.at[:, pl.ds(wid * b_per_w, b_per_w)]
        )

    return gather(*cols, idx2d)


def kernel(x_id, table, W, b):
    tableT = table.T
    cols = _tc_project_table(tableT, W, b)
    idx2d = x_id.astype(jnp.int32).reshape(-1, _CHUNK)
    pout = _sc_gather_cols(cols, idx2d)
    return pout.T


# speedup vs baseline: 1.0037x; 1.0037x over previous
import functools

import jax
import jax.numpy as jnp
from jax import lax
from jax.experimental import pallas as pl
from jax.experimental.pallas import tpu as pltpu
from jax.experimental.pallas import tpu_sc as plsc

_NC = 2
_NS = 16
_NW = _NC * _NS
_CHUNK = 128
_BLK = 32768


def _tc_project_table(tableT, W, b):
    d, v = tableT.shape
    o = W.shape[0]
    grid = (v + _BLK - 1) // _BLK

    def body(t_ref, w_ref, b_ref, *o_refs):
        res = lax.dot_general(
            w_ref[...], t_ref[...], (((1,), (0,)), ((), ())),
            preferred_element_type=jnp.float32,
        )
        for j in range(o):
            o_refs[j][...] = res[j, :] + b_ref[j]

    return pl.pallas_call(
        body,
        grid=(grid,),
        in_specs=[
            pl.BlockSpec((d, _BLK), lambda i: (0, i)),
            pl.BlockSpec((o, d), lambda i: (0, 0)),
            pl.BlockSpec(memory_space=pltpu.SMEM),
        ],
        out_specs=tuple(pl.BlockSpec((_BLK,), lambda i: (i,)) for _ in range(o)),
        out_shape=tuple(
            jax.ShapeDtypeStruct((v,), jnp.float32) for _ in range(o)
        ),
    )(tableT, W, b)


def _sc_gather_cols(cols, idx2d):
    o = len(cols)
    n_rows, chunk = idx2d.shape
    batch = n_rows * chunk
    rows_per_w = n_rows // _NW
    b_per_w = batch // _NW

    mesh = plsc.VectorSubcoreMesh(core_axis_name="c", subcore_axis_name="s")

    @functools.partial(
        pl.kernel,
        mesh=mesh,
        compiler_params=pltpu.CompilerParams(use_tc_tiling_on_sc=False),
        out_type=jax.ShapeDtypeStruct((o, batch), jnp.float32),
        scratch_types=[
            pltpu.VMEM((rows_per_w, chunk), jnp.int32),
            pltpu.VMEM((o, b_per_w), jnp.float32),
            pltpu.SemaphoreType.DMA,
        ],
    )
    def gather(*refs):
        col_hbms = refs[:o]
        idx_hbm = refs[o]
        out_hbm = refs[o + 1]
        idx_v, vals_v, sem = refs[o + 2], refs[o + 3], refs[o + 4]
        wid = lax.axis_index("s") * _NC + lax.axis_index("c")
        row_base = wid * rows_per_w
        pltpu.sync_copy(idx_hbm.at[pl.ds(row_base, rows_per_w)], idx_v)
        copies = []
        for j in range(o):
            for i in range(rows_per_w):
                copies.append(
                    pltpu.async_copy(
                        col_hbms[j].at[idx_v.at[i]],
                        vals_v.at[j, pl.ds(i * chunk, chunk)],
                        sem,
                    )
                )
        for cp in copies:
            cp.wait()
        pltpu.sync_copy(
            vals_v, out_hbm.at[:, pl.ds(wid * b_per_w, b_per_w)]
        )

    return gather(*cols, idx2d)


def kernel(x_id, table, W, b):
    tableT = table.T
    cols = _tc_project_table(tableT, W, b)
    idx2d = x_id.astype(jnp.int32).reshape(-1, _CHUNK)
    pout = _sc_gather_cols(cols, idx2d)
    return pout.T


# speedup vs baseline: 1.0145x; 1.0107x over previous
import functools

import jax
import jax.numpy as jnp
from jax import lax
from jax.experimental import pallas as pl
from jax.experimental.pallas import tpu as pltpu
from jax.experimental.pallas import tpu_sc as plsc

_NC = 2
_NS = 16
_NW = _NC * _NS
_CHUNK = 128
_BLK = 32768


def _tc_project_table(tableT, W, b):
    d, v = tableT.shape
    o = W.shape[0]
    grid = (v + _BLK - 1) // _BLK

    def body(t_ref, w_ref, b_ref, *o_refs):
        res = lax.dot_general(
            w_ref[...], t_ref[...], (((1,), (0,)), ((), ())),
            preferred_element_type=jnp.float32,
        )
        for j in range(o):
            o_refs[j][...] = res[j, :] + b_ref[j]

    return pl.pallas_call(
        body,
        grid=(grid,),
        in_specs=[
            pl.BlockSpec((d, _BLK), lambda i: (0, i)),
            pl.BlockSpec((o, d), lambda i: (0, 0)),
            pl.BlockSpec(memory_space=pltpu.SMEM),
        ],
        out_specs=tuple(pl.BlockSpec((_BLK,), lambda i: (i,)) for _ in range(o)),
        out_shape=tuple(
            jax.ShapeDtypeStruct((v,), jnp.float32) for _ in range(o)
        ),
    )(tableT, W, b)


def _sc_gather_cols(cols, idx):
    o = len(cols)
    batch = idx.shape[0]
    b_per_w = batch // _NW
    n_chunks = b_per_w // _CHUNK

    mesh = plsc.VectorSubcoreMesh(core_axis_name="c", subcore_axis_name="s")

    @functools.partial(
        pl.kernel,
        mesh=mesh,
        out_type=jax.ShapeDtypeStruct((o, batch), jnp.float32),
        scratch_types=[
            pltpu.VMEM((b_per_w,), jnp.int32),
            pltpu.VMEM((o, b_per_w), jnp.float32),
            pltpu.SemaphoreType.DMA,
        ],
    )
    def gather(*refs):
        col_hbms = refs[:o]
        idx_hbm = refs[o]
        out_hbm = refs[o + 1]
        idx_v, vals_v, sem = refs[o + 2], refs[o + 3], refs[o + 4]
        wid = lax.axis_index("s") * _NC + lax.axis_index("c")
        pltpu.sync_copy(idx_hbm.at[pl.ds(wid * b_per_w, b_per_w)], idx_v)
        copies = []
        for j in range(o):
            for i in range(n_chunks):
                copies.append(
                    pltpu.async_copy(
                        col_hbms[j].at[idx_v.at[pl.ds(i * _CHUNK, _CHUNK)]],
                        vals_v.at[j, pl.ds(i * _CHUNK, _CHUNK)],
                        sem,
                    )
                )
        for cp in copies:
            cp.wait()
        pltpu.sync_copy(
            vals_v, out_hbm.at[:, pl.ds(wid * b_per_w, b_per_w)]
        )

    return gather(*cols, idx)


def kernel(x_id, table, W, b):
    tableT = table.T
    cols = _tc_project_table(tableT, W, b)
    pout = _sc_gather_cols(cols, x_id.astype(jnp.int32))
    return pout.T
